# baseline (speedup 1.0000x reference)
import functools

import jax
import jax.numpy as jnp
from jax.experimental import pallas as pl
from jax.experimental.pallas import tpu as pltpu


def _dcn_loss_kernel(emb_ref, cen_ref, out_ref, *, inv_n):
    emb = emb_ref[...]
    cen = cen_ref[...]
    emb_bf = emb.astype(jnp.bfloat16)
    cenm2_bf = (-2.0 * cen).astype(jnp.bfloat16)
    neg2dots = jax.lax.dot_general(
        emb_bf, cenm2_bf, (((1,), (1,)), ((), ())),
        preferred_element_type=jnp.float32)
    b2 = jnp.sum(cen * cen, axis=1)[None, :]
    m = jnp.min(neg2dots + b2, axis=1)
    a2 = jnp.sum(emb * emb, axis=1)
    d = jnp.maximum(a2 + m, 0.0)
    out_ref[0, 0, 0] = jnp.sum(d) * inv_n


def kernel(embedded, centers):
    n, d = embedded.shape
    k, _ = centers.shape
    bn = 1024 if n % 1024 == 0 else n
    grid = (n // bn,)
    parts = pl.pallas_call(
        functools.partial(_dcn_loss_kernel, inv_n=1.0 / n),
        grid=grid,
        in_specs=[
            pl.BlockSpec((bn, d), lambda i: (i, 0)),
            pl.BlockSpec((k, d), lambda i: (0, 0)),
        ],
        out_specs=pl.BlockSpec((1, 1, 1), lambda i: (i, 0, 0),
                               memory_space=pltpu.SMEM),
        out_shape=jax.ShapeDtypeStruct((grid[0], 1, 1), jnp.float32),
        compiler_params=pltpu.CompilerParams(
            dimension_semantics=("parallel",)),
    )(embedded, centers)
    return jnp.sum(parts[:, 0, 0])


# speedup vs baseline: 1.2337x; 1.2337x over previous
import functools

import jax
import jax.numpy as jnp
from jax.experimental import pallas as pl
from jax.experimental.pallas import tpu as pltpu


def _dcn_loss_kernel(emb_ref, cen_ref, out_ref, *, inv_n):
    emb = emb_ref[...]
    cen = cen_ref[...]
    emb_bf = emb.astype(jnp.bfloat16)
    cenm2_bf = (-2.0 * cen).astype(jnp.bfloat16)
    neg2dots = jax.lax.dot_general(
        emb_bf, cenm2_bf, (((1,), (1,)), ((), ())),
        preferred_element_type=jnp.float32)
    b2 = jnp.sum(cen * cen, axis=1)[None, :]
    m = jnp.min(neg2dots + b2, axis=1)
    a2 = jnp.sum(emb * emb, axis=1)
    d = jnp.maximum(a2 + m, 0.0)
    out_ref[0, 0, 0] = jnp.sum(d) * inv_n


def kernel(embedded, centers):
    n, d = embedded.shape
    k, _ = centers.shape
    bn = 4096 if n % 4096 == 0 else n
    grid = (n // bn,)
    parts = pl.pallas_call(
        functools.partial(_dcn_loss_kernel, inv_n=1.0 / n),
        grid=grid,
        in_specs=[
            pl.BlockSpec((bn, d), lambda i: (i, 0)),
            pl.BlockSpec((k, d), lambda i: (0, 0)),
        ],
        out_specs=pl.BlockSpec((1, 1, 1), lambda i: (i, 0, 0),
                               memory_space=pltpu.SMEM),
        out_shape=jax.ShapeDtypeStruct((grid[0], 1, 1), jnp.float32),
        compiler_params=pltpu.CompilerParams(
            dimension_semantics=("parallel",)),
    )(embedded, centers)
    return jnp.sum(parts[:, 0, 0])
